# baseline (speedup 1.0000x reference)
import functools

import jax
import jax.numpy as jnp
from jax import lax
from jax.experimental import pallas as pl
from jax.experimental.pallas import tpu as pltpu
from jax.experimental.pallas import tpu_sc as plsc

_NEW_TOKEN_ID = 1000000
_VOCAB = 1000000
_D = 64

_NC = 2
_NS = 16
_NW = _NC * _NS

_EPC = 2


@functools.partial(jax.jit, static_argnums=(3, 4))
def _lookup(ids, table, new_row, batch, seq):
    e_per_w = batch // _NW
    per_w = e_per_w * seq
    chunk = _EPC * seq
    n_chunks = e_per_w // _EPC
    n_pairs = n_chunks // 2
    n_groups = chunk // 16
    mesh = plsc.VectorSubcoreMesh(core_axis_name="c", subcore_axis_name="s")

    @functools.partial(
        pl.kernel,
        mesh=mesh,
        out_type=jax.ShapeDtypeStruct((batch, seq, _D), jnp.float32),
        scratch_types=[
            pltpu.VMEM((per_w,), jnp.int32),
            pltpu.VMEM((chunk, _D), jnp.float32),
            pltpu.VMEM((chunk, _D), jnp.float32),
            pltpu.VMEM((_D,), jnp.float32),
            pltpu.SMEM((2,), jnp.int32),
            pltpu.SemaphoreType.DMA,
            pltpu.SemaphoreType.DMA,
            pltpu.SemaphoreType.DMA,
            pltpu.SemaphoreType.DMA,
        ],
        compiler_params=pltpu.CompilerParams(
            needs_layout_passes=False, use_tc_tiling_on_sc=False
        ),
    )
    def k(ids_hbm, table_hbm, new_hbm, out_hbm, ids_v,
          rows0, rows1, new_v, flags, gsem0, gsem1, osem0, osem1):
        wid = lax.axis_index("s") * _NC + lax.axis_index("c")
        webase = wid * e_per_w
        rows = (rows0, rows1)
        gsem = (gsem0, gsem1)
        osem = (osem0, osem1)

        pltpu.sync_copy(new_hbm, new_v)
        pltpu.sync_copy(ids_hbm.at[pl.ds(webase * seq, per_w)], ids_v)

        def pass1(ci, b):
            base = ci * chunk
            mx = None
            for g in range(n_groups):
                idv = ids_v[pl.ds(base + g * 16, 16)]
                pltpu.async_copy(
                    table_hbm.at[jnp.minimum(idv, _VOCAB - 1)],
                    rows[b].at[pl.ds(g * 16, 16)],
                    gsem[b],
                )
                mx = idv if mx is None else jnp.maximum(mx, idv)
            flags[b] = jnp.max(mx)

        def drain_gathers(b):
            pltpu.make_async_copy(
                table_hbm.at[pl.ds(0, chunk)], rows[b], gsem[b]
            ).wait()

        def out_cps(ci, b):
            e = webase + ci * _EPC
            return [
                pltpu.make_async_copy(
                    rows[b].at[pl.ds(j * seq, seq)], out_hbm.at[e + j], osem[b]
                )
                for j in range(_EPC)
            ]

        def fixup(ci, b):
            @pl.when(flags[b] >= _NEW_TOKEN_ID)
            def _fix():
                base = ci * chunk
                liota = lax.iota(jnp.int32, 16)
                for g in range(n_groups):
                    idv = ids_v[pl.ds(base + g * 16, 16)]
                    m = idv == _NEW_TOKEN_ID
                    rowv = g * 16 + liota
                    for q in range(_D // 16):
                        plsc.store_scatter(
                            rows[b],
                            [rowv, q * 16 + liota],
                            new_v[pl.ds(q * 16, 16)],
                            mask=m,
                        )

        for b in range(2):
            pass1(b, b)

        def pair(p, carry):
            for b in range(2):
                ci = 2 * p + b
                nci = ci + 2
                drain_gathers(b)
                fixup(ci, b)
                cps = out_cps(ci, b)
                for cp in cps:
                    cp.start()
                for cp in cps:
                    cp.wait()
                @pl.when(nci < n_chunks)
                def _prep():
                    pass1(nci, b)
            return carry

        lax.fori_loop(0, n_pairs, pair, 0)

    return k(ids, table, new_row)


def kernel(input_ids, base_weight, new_weight):
    b, s = input_ids.shape
    ids = input_ids.reshape(b * s).astype(jnp.int32)
    return _lookup(ids, base_weight, new_weight.reshape(_D), b, s)
